# baseline (speedup 1.0000x reference)
import functools

import jax
import jax.numpy as jnp
from jax import lax
from jax.experimental import pallas as pl
from jax.experimental.pallas import tpu as pltpu
from jax.experimental.pallas import tpu_sc as plsc

_AVG_RATING = 3.5
_NUM_CORES = 2
_NUM_SUBCORES = 16
_NW = _NUM_CORES * _NUM_SUBCORES


def _pack_body(guT, muT, giT, miT, eg, em, out_u, out_i):
    c00 = (((0,), (0,)), ((), ()))
    prec = lax.Precision.HIGHEST
    f32 = jnp.float32
    out_u[...] = (
        lax.dot_general(guT[...], eg[...], c00, precision=prec,
                        preferred_element_type=f32)
        + lax.dot_general(muT[...], em[...], c00, precision=prec,
                          preferred_element_type=f32))
    out_i[...] = (
        lax.dot_general(giT[...], eg[...], c00, precision=prec,
                        preferred_element_type=f32)
        + lax.dot_general(miT[...], em[...], c00, precision=prec,
                          preferred_element_type=f32))


def _pack_pairs(guT, muT, giT, miT):
    V = guT.shape[1]
    D = guT.shape[0]
    nb = 3200
    grid = (pl.cdiv(V, nb),)
    eye = jnp.eye(D, dtype=jnp.float32)
    zero = jnp.zeros((D, D), dtype=jnp.float32)
    eg = jnp.concatenate([eye, zero], axis=1)
    em = jnp.concatenate([zero, eye], axis=1)
    tab_spec = pl.BlockSpec((D, nb), lambda i: (0, i))
    e_spec = pl.BlockSpec((D, 2 * D), lambda i: (0, 0))
    out_spec = pl.BlockSpec((nb, 2 * D), lambda i: (i, 0))
    return pl.pallas_call(
        _pack_body,
        grid=grid,
        in_specs=[tab_spec, tab_spec, tab_spec, tab_spec, e_spec, e_spec],
        out_specs=[out_spec, out_spec],
        out_shape=[jax.ShapeDtypeStruct((V, 2 * D), jnp.float32)] * 2,
    )(guT, muT, giT, miT, eg, em)


def _sc_gather2(user, item, user_tab, item_tab):
    B = user.shape[0]
    D = user_tab.shape[1]
    bpw = B // _NW
    f32 = jnp.float32
    mesh = plsc.VectorSubcoreMesh(core_axis_name="c", subcore_axis_name="s")

    @functools.partial(
        pl.kernel,
        mesh=mesh,
        out_type=(jax.ShapeDtypeStruct((B, D), f32),
                  jax.ShapeDtypeStruct((B, D), f32)),
        scratch_types=[
            pltpu.VMEM((bpw,), jnp.int32),
            pltpu.VMEM((bpw,), jnp.int32),
            pltpu.VMEM((bpw, D), f32),
            pltpu.VMEM((bpw, D), f32),
            pltpu.SemaphoreType.DMA,
            pltpu.SemaphoreType.DMA,
        ],
    )
    def gather_kernel(u_hbm, i_hbm, ut, it, o0, o1, iu, ii, r0, r1, s0, s1):
        wid = lax.axis_index("s") * _NUM_CORES + lax.axis_index("c")
        base = wid * bpw
        pltpu.sync_copy(u_hbm.at[pl.ds(base, bpw)], iu)
        pltpu.sync_copy(i_hbm.at[pl.ds(base, bpw)], ii)
        c0 = pltpu.async_copy(ut.at[iu], r0, s0)
        c1 = pltpu.async_copy(it.at[ii], r1, s1)
        c0.wait()
        pltpu.sync_copy(r0, o0.at[pl.ds(base, bpw)])
        c1.wait()
        pltpu.sync_copy(r1, o1.at[pl.ds(base, bpw)])

    return gather_kernel(user, item, user_tab, item_tab)


def _dense_body(ur, ir, w1, w2, wf, bf, lab, pred_o, obj_o, mse_o):
    dim = ur.shape[1] // 2
    dn = (((1,), (0,)), ((), ()))
    prec = lax.Precision.DEFAULT
    gu = ur[:, 0:dim]
    mu = ur[:, dim:2 * dim]
    gi = ir[:, 0:dim]
    mi = ir[:, dim:2 * dim]
    h = lax.dot_general(mu, w1[0:dim, :], dn, precision=prec,
                        preferred_element_type=jnp.float32)
    h += lax.dot_general(mi, w1[dim:2 * dim, :], dn, precision=prec,
                         preferred_element_type=jnp.float32)
    h = jnp.maximum(h, 0.0)
    h = lax.dot_general(h, w2[...], dn, precision=prec,
                        preferred_element_type=jnp.float32)
    h = jnp.maximum(h, 0.0)
    g = gu * gi
    pred = lax.dot_general(g, wf[0:dim, :], dn, precision=prec,
                           preferred_element_type=jnp.float32)
    pred += lax.dot_general(h, wf[dim:2 * dim, :], dn, precision=prec,
                            preferred_element_type=jnp.float32)
    pred = pred + (bf[0, 0] + _AVG_RATING)
    diff = pred - lab[...]
    mse = diff * diff
    pred_o[...] = pred
    mse_o[...] = mse
    obj_o[...] = jnp.sum(mse).reshape(1, 1)


def _dense(ur, ir, W1, W2, Wf, bf, label):
    B = ur.shape[0]
    f32 = jnp.float32
    out_shape = [
        jax.ShapeDtypeStruct((B, 1), f32),
        jax.ShapeDtypeStruct((1, 1), f32),
        jax.ShapeDtypeStruct((B, 1), f32),
    ]
    return pl.pallas_call(_dense_body, out_shape=out_shape)(
        ur, ir, W1, W2, Wf, bf.reshape(1, 1), label.reshape(B, 1))


def kernel(user, item, label, gmf_user_table, gmf_item_table, mlp_user_table,
           mlp_item_table, W1, W2, Wf, bf):
    user = user.astype(jnp.int32)
    item = item.astype(jnp.int32)
    user_tab, item_tab = _pack_pairs(
        gmf_user_table.T, mlp_user_table.T, gmf_item_table.T,
        mlp_item_table.T)
    ur, ir = _sc_gather2(user, item, user_tab, item_tab)
    pred, obj, mse = _dense(ur, ir, W1, W2, Wf, bf, label)
    return pred.reshape(-1), obj.reshape(()), mse.reshape(-1)


# speedup vs baseline: 2.2613x; 2.2613x over previous
import functools

import jax
import jax.numpy as jnp
from jax import lax
from jax.experimental import pallas as pl
from jax.experimental.pallas import tpu as pltpu
from jax.experimental.pallas import tpu_sc as plsc

_AVG_RATING = 3.5
_NUM_CORES = 2
_NUM_SUBCORES = 16
_NW = _NUM_CORES * _NUM_SUBCORES


def _pack_body(guT, muT, giT, miT, out_u, out_i):
    out_u[...] = jnp.concatenate([guT[...], muT[...]], axis=0).T
    out_i[...] = jnp.concatenate([giT[...], miT[...]], axis=0).T


def _pack_pairs(guT, muT, giT, miT):
    V = guT.shape[1]
    D = guT.shape[0]
    nb = 3200
    grid = (pl.cdiv(V, nb),)
    tab_spec = pl.BlockSpec((D, nb), lambda i: (0, i))
    out_spec = pl.BlockSpec((nb, 2 * D), lambda i: (i, 0))
    return pl.pallas_call(
        _pack_body,
        grid=grid,
        in_specs=[tab_spec, tab_spec, tab_spec, tab_spec],
        out_specs=[out_spec, out_spec],
        out_shape=[jax.ShapeDtypeStruct((V, 2 * D), jnp.float32)] * 2,
    )(guT, muT, giT, miT)


def _sc_gather2(user, item, user_tab, item_tab):
    B = user.shape[0]
    D = user_tab.shape[1]
    bpw = B // _NW
    f32 = jnp.float32
    mesh = plsc.VectorSubcoreMesh(core_axis_name="c", subcore_axis_name="s")

    @functools.partial(
        pl.kernel,
        mesh=mesh,
        out_type=(jax.ShapeDtypeStruct((B, D), f32),
                  jax.ShapeDtypeStruct((B, D), f32)),
        scratch_types=[
            pltpu.VMEM((bpw,), jnp.int32),
            pltpu.VMEM((bpw,), jnp.int32),
            pltpu.VMEM((bpw, D), f32),
            pltpu.VMEM((bpw, D), f32),
            pltpu.SemaphoreType.DMA,
            pltpu.SemaphoreType.DMA,
        ],
    )
    def gather_kernel(u_hbm, i_hbm, ut, it, o0, o1, iu, ii, r0, r1, s0, s1):
        wid = lax.axis_index("s") * _NUM_CORES + lax.axis_index("c")
        base = wid * bpw
        pltpu.sync_copy(u_hbm.at[pl.ds(base, bpw)], iu)
        pltpu.sync_copy(i_hbm.at[pl.ds(base, bpw)], ii)
        c0 = pltpu.async_copy(ut.at[iu], r0, s0)
        c1 = pltpu.async_copy(it.at[ii], r1, s1)
        c0.wait()
        pltpu.sync_copy(r0, o0.at[pl.ds(base, bpw)])
        c1.wait()
        pltpu.sync_copy(r1, o1.at[pl.ds(base, bpw)])

    return gather_kernel(user, item, user_tab, item_tab)


def _dense_body(ur, ir, w1, w2, wf, bf, lab, pred_o, obj_o, mse_o):
    dim = ur.shape[1] // 2
    dn = (((1,), (0,)), ((), ()))
    prec = lax.Precision.DEFAULT
    gu = ur[:, 0:dim]
    mu = ur[:, dim:2 * dim]
    gi = ir[:, 0:dim]
    mi = ir[:, dim:2 * dim]
    h = lax.dot_general(mu, w1[0:dim, :], dn, precision=prec,
                        preferred_element_type=jnp.float32)
    h += lax.dot_general(mi, w1[dim:2 * dim, :], dn, precision=prec,
                         preferred_element_type=jnp.float32)
    h = jnp.maximum(h, 0.0)
    h = lax.dot_general(h, w2[...], dn, precision=prec,
                        preferred_element_type=jnp.float32)
    h = jnp.maximum(h, 0.0)
    g = gu * gi
    pred = lax.dot_general(g, wf[0:dim, :], dn, precision=prec,
                           preferred_element_type=jnp.float32)
    pred += lax.dot_general(h, wf[dim:2 * dim, :], dn, precision=prec,
                            preferred_element_type=jnp.float32)
    pred = pred + (bf[0, 0] + _AVG_RATING)
    diff = pred - lab[...]
    mse = diff * diff
    pred_o[...] = pred
    mse_o[...] = mse
    obj_o[...] = jnp.sum(mse).reshape(1, 1)


def _dense(ur, ir, W1, W2, Wf, bf, label):
    B = ur.shape[0]
    f32 = jnp.float32
    out_shape = [
        jax.ShapeDtypeStruct((B, 1), f32),
        jax.ShapeDtypeStruct((1, 1), f32),
        jax.ShapeDtypeStruct((B, 1), f32),
    ]
    return pl.pallas_call(_dense_body, out_shape=out_shape)(
        ur, ir, W1, W2, Wf, bf.reshape(1, 1), label.reshape(B, 1))


def kernel(user, item, label, gmf_user_table, gmf_item_table, mlp_user_table,
           mlp_item_table, W1, W2, Wf, bf):
    user = user.astype(jnp.int32)
    item = item.astype(jnp.int32)
    user_tab, item_tab = _pack_pairs(
        gmf_user_table.T, mlp_user_table.T, gmf_item_table.T,
        mlp_item_table.T)
    ur, ir = _sc_gather2(user, item, user_tab, item_tab)
    pred, obj, mse = _dense(ur, ir, W1, W2, Wf, bf, label)
    return pred.reshape(-1), obj.reshape(()), mse.reshape(-1)


# speedup vs baseline: 2.3316x; 1.0311x over previous
import functools

import jax
import jax.numpy as jnp
from jax import lax
from jax.experimental import pallas as pl
from jax.experimental.pallas import tpu as pltpu
from jax.experimental.pallas import tpu_sc as plsc

_AVG_RATING = 3.5
_NUM_CORES = 2
_NUM_SUBCORES = 16
_NW = _NUM_CORES * _NUM_SUBCORES


def _bits32(x_f32):
    n, m = x_f32.shape
    k = m // 2
    u16 = lax.bitcast_convert_type(x_f32.astype(jnp.bfloat16), jnp.uint16)
    u32 = u16.astype(jnp.uint32)
    packed = lax.bitwise_or(u32[:, 0:k],
                            lax.shift_left(u32[:, k:m], jnp.uint32(16)))
    return lax.bitcast_convert_type(packed, jnp.float32)


def _pack_body(guT, muT, giT, miT, out):
    u = jnp.concatenate([guT[...], muT[...]], axis=0).T
    i = jnp.concatenate([giT[...], miT[...]], axis=0).T
    out[...] = jnp.concatenate([_bits32(u), _bits32(i)], axis=1)


def _pack_pairs(guT, muT, giT, miT):
    V = guT.shape[1]
    D = guT.shape[0]
    nb = 3200
    grid = (pl.cdiv(V, nb),)
    tab_spec = pl.BlockSpec((D, nb), lambda i: (0, i))
    out_spec = pl.BlockSpec((nb, 2 * D), lambda i: (i, 0))
    return pl.pallas_call(
        _pack_body,
        grid=grid,
        in_specs=[tab_spec, tab_spec, tab_spec, tab_spec],
        out_specs=out_spec,
        out_shape=jax.ShapeDtypeStruct((V, 2 * D), jnp.float32),
    )(guT, muT, giT, miT)


def _sc_gather2(user, item, tab):
    B = user.shape[0]
    D = tab.shape[1]
    bpw = B // _NW
    f32 = jnp.float32
    mesh = plsc.VectorSubcoreMesh(core_axis_name="c", subcore_axis_name="s")

    @functools.partial(
        pl.kernel,
        mesh=mesh,
        out_type=(jax.ShapeDtypeStruct((B, D), f32),
                  jax.ShapeDtypeStruct((B, D), f32)),
        scratch_types=[
            pltpu.VMEM((bpw,), jnp.int32),
            pltpu.VMEM((bpw,), jnp.int32),
            pltpu.VMEM((bpw, D), f32),
            pltpu.VMEM((bpw, D), f32),
            pltpu.SemaphoreType.DMA,
            pltpu.SemaphoreType.DMA,
        ],
    )
    def gather_kernel(u_hbm, i_hbm, t_hbm, o0, o1, iu, ii, r0, r1, s0, s1):
        wid = lax.axis_index("s") * _NUM_CORES + lax.axis_index("c")
        base = wid * bpw
        pltpu.sync_copy(u_hbm.at[pl.ds(base, bpw)], iu)
        pltpu.sync_copy(i_hbm.at[pl.ds(base, bpw)], ii)
        c0 = pltpu.async_copy(t_hbm.at[iu], r0, s0)
        c1 = pltpu.async_copy(t_hbm.at[ii], r1, s1)
        c0.wait()
        pltpu.sync_copy(r0, o0.at[pl.ds(base, bpw)])
        c1.wait()
        pltpu.sync_copy(r1, o1.at[pl.ds(base, bpw)])

    return gather_kernel(user, item, tab)


def _unbits(x_f32):
    u32 = lax.bitcast_convert_type(x_f32, jnp.uint32)
    lo = lax.bitcast_convert_type(u32.astype(jnp.uint16), jnp.bfloat16)
    hi = lax.bitcast_convert_type(
        lax.shift_right_logical(u32, jnp.uint32(16)).astype(jnp.uint16),
        jnp.bfloat16)
    return lo, hi


def _dense_body(ug, ig, w1, w2, wf, bf, lab, pred_o, obj_o, mse_o):
    dim = ug.shape[1] // 2
    dn = (((1,), (0,)), ((), ()))
    prec = lax.Precision.DEFAULT
    f32 = jnp.float32
    gu, mu = _unbits(ug[:, 0:dim])
    gi, mi = _unbits(ig[:, dim:2 * dim])
    gu = gu.astype(f32)
    gi = gi.astype(f32)
    h = lax.dot_general(mu, w1[0:dim, :], dn, precision=prec,
                        preferred_element_type=f32)
    h += lax.dot_general(mi, w1[dim:2 * dim, :], dn, precision=prec,
                         preferred_element_type=f32)
    h = jnp.maximum(h, 0.0)
    h = lax.dot_general(h, w2[...], dn, precision=prec,
                        preferred_element_type=f32)
    h = jnp.maximum(h, 0.0)
    g = gu * gi
    pred = lax.dot_general(g, wf[0:dim, :], dn, precision=prec,
                           preferred_element_type=f32)
    pred += lax.dot_general(h, wf[dim:2 * dim, :], dn, precision=prec,
                            preferred_element_type=f32)
    pred = pred + (bf[0, 0] + _AVG_RATING)
    diff = pred - lab[...]
    mse = diff * diff
    pred_o[...] = pred
    mse_o[...] = mse
    obj_o[...] = jnp.sum(mse).reshape(1, 1)


def _dense(ug, ig, W1, W2, Wf, bf, label):
    B = ug.shape[0]
    f32 = jnp.float32
    out_shape = [
        jax.ShapeDtypeStruct((B, 1), f32),
        jax.ShapeDtypeStruct((1, 1), f32),
        jax.ShapeDtypeStruct((B, 1), f32),
    ]
    return pl.pallas_call(_dense_body, out_shape=out_shape)(
        ug, ig, W1, W2, Wf, bf.reshape(1, 1), label.reshape(B, 1))


def kernel(user, item, label, gmf_user_table, gmf_item_table, mlp_user_table,
           mlp_item_table, W1, W2, Wf, bf):
    user = user.astype(jnp.int32)
    item = item.astype(jnp.int32)
    tab = _pack_pairs(gmf_user_table.T, mlp_user_table.T, gmf_item_table.T,
                      mlp_item_table.T)
    ug, ig = _sc_gather2(user, item, tab)
    pred, obj, mse = _dense(ug, ig, W1, W2, Wf, bf, label)
    return pred.reshape(-1), obj.reshape(()), mse.reshape(-1)


# speedup vs baseline: 2.5419x; 1.0902x over previous
import functools

import jax
import jax.numpy as jnp
from jax import lax
from jax.experimental import pallas as pl
from jax.experimental.pallas import tpu as pltpu
from jax.experimental.pallas import tpu_sc as plsc

_AVG_RATING = 3.5
_NUM_CORES = 2
_NUM_SUBCORES = 16
_NW = _NUM_CORES * _NUM_SUBCORES


def _bits32(x_f32):
    n, m = x_f32.shape
    k = m // 2
    u16 = lax.bitcast_convert_type(x_f32.astype(jnp.bfloat16), jnp.uint16)
    u32 = u16.astype(jnp.uint32)
    packed = lax.bitwise_or(u32[:, 0:k],
                            lax.shift_left(u32[:, k:m], jnp.uint32(16)))
    return lax.bitcast_convert_type(packed, jnp.float32)


def _pack_body(guT, muT, giT, miT, out):
    u = jnp.concatenate([guT[...], muT[...]], axis=0).T
    i = jnp.concatenate([giT[...], miT[...]], axis=0).T
    out[...] = jnp.concatenate([_bits32(u), _bits32(i)], axis=1)


def _pack_pairs(guT, muT, giT, miT):
    V = guT.shape[1]
    D = guT.shape[0]
    nb = 6400
    grid = (pl.cdiv(V, nb),)
    tab_spec = pl.BlockSpec((D, nb), lambda i: (0, i))
    out_spec = pl.BlockSpec((nb, 2 * D), lambda i: (i, 0))
    return pl.pallas_call(
        _pack_body,
        grid=grid,
        in_specs=[tab_spec, tab_spec, tab_spec, tab_spec],
        out_specs=out_spec,
        out_shape=jax.ShapeDtypeStruct((V, 2 * D), jnp.float32),
    )(guT, muT, giT, miT)


def _sc_gather2(user, item, tab):
    B = user.shape[0]
    D = tab.shape[1]
    bpw = B // _NW
    f32 = jnp.float32
    mesh = plsc.VectorSubcoreMesh(core_axis_name="c", subcore_axis_name="s")

    @functools.partial(
        pl.kernel,
        mesh=mesh,
        out_type=(jax.ShapeDtypeStruct((B, D), f32),
                  jax.ShapeDtypeStruct((B, D), f32)),
        scratch_types=[
            pltpu.VMEM((bpw,), jnp.int32),
            pltpu.VMEM((bpw,), jnp.int32),
            pltpu.VMEM((bpw, D), f32),
            pltpu.VMEM((bpw, D), f32),
            pltpu.SemaphoreType.DMA,
            pltpu.SemaphoreType.DMA,
        ],
    )
    def gather_kernel(u_hbm, i_hbm, t_hbm, o0, o1, iu, ii, r0, r1, s0, s1):
        wid = lax.axis_index("s") * _NUM_CORES + lax.axis_index("c")
        base = wid * bpw
        pltpu.sync_copy(u_hbm.at[pl.ds(base, bpw)], iu)
        pltpu.sync_copy(i_hbm.at[pl.ds(base, bpw)], ii)
        c0 = pltpu.async_copy(t_hbm.at[iu], r0, s0)
        c1 = pltpu.async_copy(t_hbm.at[ii], r1, s1)
        c0.wait()
        pltpu.sync_copy(r0, o0.at[pl.ds(base, bpw)])
        c1.wait()
        pltpu.sync_copy(r1, o1.at[pl.ds(base, bpw)])

    return gather_kernel(user, item, tab)


def _unbits(x_f32):
    u32 = lax.bitcast_convert_type(x_f32, jnp.uint32)
    lo = lax.bitcast_convert_type(u32.astype(jnp.uint16), jnp.bfloat16)
    hi = lax.bitcast_convert_type(
        lax.shift_right_logical(u32, jnp.uint32(16)).astype(jnp.uint16),
        jnp.bfloat16)
    return lo, hi


def _dense_body(ug, ig, w1, w2, wf, bf, lab, pred_o, obj_o, mse_o):
    dim = ug.shape[1] // 2
    dn = (((1,), (0,)), ((), ()))
    prec = lax.Precision.DEFAULT
    f32 = jnp.float32
    gu, mu = _unbits(ug[:, 0:dim])
    gi, mi = _unbits(ig[:, dim:2 * dim])
    gu = gu.astype(f32)
    gi = gi.astype(f32)
    h = lax.dot_general(mu, w1[0:dim, :], dn, precision=prec,
                        preferred_element_type=f32)
    h += lax.dot_general(mi, w1[dim:2 * dim, :], dn, precision=prec,
                         preferred_element_type=f32)
    h = jnp.maximum(h, 0.0)
    h = lax.dot_general(h, w2[...], dn, precision=prec,
                        preferred_element_type=f32)
    h = jnp.maximum(h, 0.0)
    g = gu * gi
    pred = lax.dot_general(g, wf[0:dim, :], dn, precision=prec,
                           preferred_element_type=f32)
    pred += lax.dot_general(h, wf[dim:2 * dim, :], dn, precision=prec,
                            preferred_element_type=f32)
    pred = pred + (bf[0, 0] + _AVG_RATING)
    diff = pred - lab[...]
    mse = diff * diff
    pred_o[...] = pred
    mse_o[...] = mse
    obj_o[...] = jnp.sum(mse).reshape(1, 1)


def _dense(ug, ig, W1, W2, Wf, bf, label):
    B = ug.shape[0]
    f32 = jnp.float32
    out_shape = [
        jax.ShapeDtypeStruct((B, 1), f32),
        jax.ShapeDtypeStruct((1, 1), f32),
        jax.ShapeDtypeStruct((B, 1), f32),
    ]
    return pl.pallas_call(_dense_body, out_shape=out_shape)(
        ug, ig, W1, W2, Wf, bf.reshape(1, 1), label.reshape(B, 1))


def kernel(user, item, label, gmf_user_table, gmf_item_table, mlp_user_table,
           mlp_item_table, W1, W2, Wf, bf):
    user = user.astype(jnp.int32)
    item = item.astype(jnp.int32)
    tab = _pack_pairs(gmf_user_table.T, mlp_user_table.T, gmf_item_table.T,
                      mlp_item_table.T)
    ug, ig = _sc_gather2(user, item, tab)
    pred, obj, mse = _dense(ug, ig, W1, W2, Wf, bf, label)
    return pred.reshape(-1), obj.reshape(()), mse.reshape(-1)


# speedup vs baseline: 2.5596x; 1.0069x over previous
import functools

import jax
import jax.numpy as jnp
from jax import lax
from jax.experimental import pallas as pl
from jax.experimental.pallas import tpu as pltpu
from jax.experimental.pallas import tpu_sc as plsc

_AVG_RATING = 3.5
_NUM_CORES = 2
_NUM_SUBCORES = 16
_NW = _NUM_CORES * _NUM_SUBCORES


def _bits32(x_f32):
    n, m = x_f32.shape
    k = m // 2
    u16 = lax.bitcast_convert_type(x_f32.astype(jnp.bfloat16), jnp.uint16)
    u32 = u16.astype(jnp.uint32)
    packed = lax.bitwise_or(u32[:, 0:k],
                            lax.shift_left(u32[:, k:m], jnp.uint32(16)))
    return lax.bitcast_convert_type(packed, jnp.float32)


def _pack_body(guT, muT, giT, miT, out):
    u = jnp.concatenate([guT[...], muT[...]], axis=0).T
    i = jnp.concatenate([giT[...], miT[...]], axis=0).T
    out[...] = jnp.concatenate([_bits32(u), _bits32(i)], axis=1)


def _pack_pairs(guT, muT, giT, miT):
    V = guT.shape[1]
    D = guT.shape[0]
    nb = 12800
    grid = (pl.cdiv(V, nb),)
    tab_spec = pl.BlockSpec((D, nb), lambda i: (0, i))
    out_spec = pl.BlockSpec((nb, 2 * D), lambda i: (i, 0))
    return pl.pallas_call(
        _pack_body,
        grid=grid,
        in_specs=[tab_spec, tab_spec, tab_spec, tab_spec],
        out_specs=out_spec,
        out_shape=jax.ShapeDtypeStruct((V, 2 * D), jnp.float32),
    )(guT, muT, giT, miT)


def _sc_gather2(user, item, tab):
    B = user.shape[0]
    D = tab.shape[1]
    bpw = B // _NW
    f32 = jnp.float32
    mesh = plsc.VectorSubcoreMesh(core_axis_name="c", subcore_axis_name="s")

    @functools.partial(
        pl.kernel,
        mesh=mesh,
        out_type=(jax.ShapeDtypeStruct((B, D), f32),
                  jax.ShapeDtypeStruct((B, D), f32)),
        scratch_types=[
            pltpu.VMEM((bpw,), jnp.int32),
            pltpu.VMEM((bpw,), jnp.int32),
            pltpu.VMEM((bpw, D), f32),
            pltpu.VMEM((bpw, D), f32),
            pltpu.SemaphoreType.DMA,
            pltpu.SemaphoreType.DMA,
        ],
    )
    def gather_kernel(u_hbm, i_hbm, t_hbm, o0, o1, iu, ii, r0, r1, s0, s1):
        wid = lax.axis_index("s") * _NUM_CORES + lax.axis_index("c")
        base = wid * bpw
        pltpu.sync_copy(u_hbm.at[pl.ds(base, bpw)], iu)
        pltpu.sync_copy(i_hbm.at[pl.ds(base, bpw)], ii)
        c0 = pltpu.async_copy(t_hbm.at[iu], r0, s0)
        c1 = pltpu.async_copy(t_hbm.at[ii], r1, s1)
        c0.wait()
        pltpu.sync_copy(r0, o0.at[pl.ds(base, bpw)])
        c1.wait()
        pltpu.sync_copy(r1, o1.at[pl.ds(base, bpw)])

    return gather_kernel(user, item, tab)


def _unbits(x_f32):
    u32 = lax.bitcast_convert_type(x_f32, jnp.uint32)
    lo = lax.bitcast_convert_type(u32.astype(jnp.uint16), jnp.bfloat16)
    hi = lax.bitcast_convert_type(
        lax.shift_right_logical(u32, jnp.uint32(16)).astype(jnp.uint16),
        jnp.bfloat16)
    return lo, hi


def _dense_body(ug, ig, w1, w2, wf, bf, lab, pred_o, obj_o, mse_o):
    dim = ug.shape[1] // 2
    dn = (((1,), (0,)), ((), ()))
    prec = lax.Precision.DEFAULT
    f32 = jnp.float32
    gu, mu = _unbits(ug[:, 0:dim])
    gi, mi = _unbits(ig[:, dim:2 * dim])
    gu = gu.astype(f32)
    gi = gi.astype(f32)
    h = lax.dot_general(mu, w1[0:dim, :], dn, precision=prec,
                        preferred_element_type=f32)
    h += lax.dot_general(mi, w1[dim:2 * dim, :], dn, precision=prec,
                         preferred_element_type=f32)
    h = jnp.maximum(h, 0.0)
    h = lax.dot_general(h, w2[...], dn, precision=prec,
                        preferred_element_type=f32)
    h = jnp.maximum(h, 0.0)
    g = gu * gi
    pred = lax.dot_general(g, wf[0:dim, :], dn, precision=prec,
                           preferred_element_type=f32)
    pred += lax.dot_general(h, wf[dim:2 * dim, :], dn, precision=prec,
                            preferred_element_type=f32)
    pred = pred + (bf[0, 0] + _AVG_RATING)
    diff = pred - lab[...]
    mse = diff * diff
    pred_o[...] = pred
    mse_o[...] = mse
    obj_o[...] = jnp.sum(mse).reshape(1, 1)


def _dense(ug, ig, W1, W2, Wf, bf, label):
    B = ug.shape[0]
    f32 = jnp.float32
    out_shape = [
        jax.ShapeDtypeStruct((B, 1), f32),
        jax.ShapeDtypeStruct((1, 1), f32),
        jax.ShapeDtypeStruct((B, 1), f32),
    ]
    return pl.pallas_call(_dense_body, out_shape=out_shape)(
        ug, ig, W1, W2, Wf, bf.reshape(1, 1), label.reshape(B, 1))


def kernel(user, item, label, gmf_user_table, gmf_item_table, mlp_user_table,
           mlp_item_table, W1, W2, Wf, bf):
    user = user.astype(jnp.int32)
    item = item.astype(jnp.int32)
    tab = _pack_pairs(gmf_user_table.T, mlp_user_table.T, gmf_item_table.T,
                      mlp_item_table.T)
    ug, ig = _sc_gather2(user, item, tab)
    pred, obj, mse = _dense(ug, ig, W1, W2, Wf, bf, label)
    return pred.reshape(-1), obj.reshape(()), mse.reshape(-1)
